# baseline (speedup 1.0000x reference)
import functools

import jax
import jax.numpy as jnp
from jax import lax
from jax.experimental import pallas as pl
from jax.experimental.pallas import tpu as pltpu
from jax.experimental.pallas import tpu_sc as plsc

_L = 16
_CHUNK = 128


def _rotate(x, k):
    perm = (lax.iota(jnp.int32, _L) + k) & (_L - 1)
    return lax.gather(
        x, perm[:, None],
        dimension_numbers=lax.GatherDimensionNumbers(
            offset_dims=(), collapsed_slice_dims=(0,), start_index_map=(0,)),
        slice_sizes=(1,),
        mode=lax.GatherScatterMode.PROMISE_IN_BOUNDS)


@functools.lru_cache(maxsize=None)
def _build(B, D):
    info = plsc.get_sparse_core_info()
    nc, ns = info.num_cores, info.num_subcores
    nw = nc * ns
    bpw = B // nw
    nchunks = bpw // _CHUNK
    mesh = plsc.VectorSubcoreMesh(core_axis_name="c", subcore_axis_name="s")

    @functools.partial(
        pl.kernel,
        mesh=mesh,
        compiler_params=pltpu.CompilerParams(use_tc_tiling_on_sc=False),
        out_type=[
            jax.ShapeDtypeStruct((B,), jnp.float32),
            jax.ShapeDtypeStruct((B, D), jnp.float32),
            jax.ShapeDtypeStruct((B, D), jnp.float32),
            jax.ShapeDtypeStruct((B,), jnp.float32),
            jax.ShapeDtypeStruct((B,), jnp.float32),
        ],
        scratch_types=[
            pltpu.VMEM((nchunks, _CHUNK), jnp.int32),
            pltpu.VMEM((nchunks, _CHUNK), jnp.int32),
            pltpu.VMEM((bpw, D), jnp.float32),
            pltpu.VMEM((bpw, D), jnp.float32),
            pltpu.VMEM((bpw,), jnp.float32),
            pltpu.VMEM((bpw,), jnp.float32),
            pltpu.VMEM((bpw,), jnp.float32),
            pltpu.SemaphoreType.DMA,
        ],
    )
    def mf_kernel(q_hbm, u_hbm, ufac_hbm, qfac_hbm, ubias_hbm, qbias_hbm,
                  res_out, uf_out, qf_out, ub_out, qb_out,
                  uidx_v, qidx_v, uf_v, qf_v, ub_v, qb_v, res_v, sem):
        wid = lax.axis_index("s") * nc + lax.axis_index("c")
        base = wid * bpw

        pltpu.sync_copy(u_hbm.at[pl.ds(wid * nchunks, nchunks)], uidx_v)
        pltpu.sync_copy(q_hbm.at[pl.ds(wid * nchunks, nchunks)], qidx_v)

        copies = []
        for j in range(nchunks):
            sl = pl.ds(j * _CHUNK, _CHUNK)
            copies.append(pltpu.async_copy(ufac_hbm.at[uidx_v.at[j]], uf_v.at[sl], sem))
            copies.append(pltpu.async_copy(qfac_hbm.at[qidx_v.at[j]], qf_v.at[sl], sem))
            copies.append(pltpu.async_copy(ubias_hbm.at[uidx_v.at[j]], ub_v.at[sl], sem))
            copies.append(pltpu.async_copy(qbias_hbm.at[qidx_v.at[j]], qb_v.at[sl], sem))
        for c in copies:
            c.wait()

        nvec = D // _L

        def body(i, carry):
            lane = lax.iota(jnp.int32, _L)
            r0 = i * _L
            acc = jnp.zeros((_L,), jnp.float32)
            for j in range(_L):
                s = None
                for v in range(nvec):
                    a = uf_v[r0 + j, pl.ds(v * _L, _L)]
                    b = qf_v[r0 + j, pl.ds(v * _L, _L)]
                    p = a * b
                    s = p if s is None else s + p
                for k in (8, 4, 2, 1):
                    s = s + _rotate(s, k)
                acc = jnp.where(lane == j, s, acc)
            x = acc + ub_v[pl.ds(r0, _L)] + qb_v[pl.ds(r0, _L)]
            res_v[pl.ds(r0, _L)] = 1.0 / (1.0 + jnp.exp(-x))
            return carry

        lax.fori_loop(0, bpw // _L, body, 0)

        pltpu.sync_copy(res_v, res_out.at[pl.ds(base, bpw)])
        pltpu.sync_copy(uf_v, uf_out.at[pl.ds(base, bpw)])
        pltpu.sync_copy(qf_v, qf_out.at[pl.ds(base, bpw)])
        pltpu.sync_copy(ub_v, ub_out.at[pl.ds(base, bpw)])
        pltpu.sync_copy(qb_v, qb_out.at[pl.ds(base, bpw)])

    return mf_kernel


def kernel(question, user, user_factors, question_factors, user_biases, question_biases):
    B = question.shape[0]
    D = user_factors.shape[1]
    q2 = question.astype(jnp.int32).reshape(B // _CHUNK, _CHUNK)
    u2 = user.astype(jnp.int32).reshape(B // _CHUNK, _CHUNK)
    res, uf, qf, ub, qb = _build(B, D)(
        q2, u2, user_factors, question_factors,
        user_biases.reshape(-1), question_biases.reshape(-1))
    return (res.reshape(B, 1), uf, qf, ub.reshape(B, 1), qb.reshape(B, 1))
